# baseline (speedup 1.0000x reference)
import jax
import jax.numpy as jnp
from jax.experimental import pallas as pl

HEAD_DIM = 128
BLOCK_DIM = 8
N_BLOCKS = HEAD_DIM // BLOCK_DIM
N_CENTROIDS = 256
NC_ALL = N_BLOCKS * N_CENTROIDS

_DEFAULT = jax.lax.Precision.DEFAULT
_CHUNKS = 2


def _vq_body(kf_ref, we_ref, wcn_ref, wda_ref, c2_ref, idx_ref, khat_ref):
    we = we_ref[...]
    wcn = wcn_ref[...]
    wda = wda_ref[...]
    c2 = c2_ref[...]
    t = kf_ref.shape[0]
    ch = t // _CHUNKS
    for s in range(_CHUNKS):
        rows = pl.ds(s * ch, ch)
        ks = jax.lax.dot_general(
            kf_ref[rows, :].astype(jnp.bfloat16), we, (((1,), (0,)), ((), ())),
            precision=_DEFAULT, preferred_element_type=jnp.float32)
        crossn = jax.lax.dot_general(
            ks.astype(jnp.bfloat16), wcn, (((1,), (0,)), ((), ())),
            precision=_DEFAULT, preferred_element_type=jnp.float32)
        d2 = c2 + crossn
        hot_parts = []
        for b in range(N_BLOCKS):
            d2_b = d2[:, b * N_CENTROIDS:(b + 1) * N_CENTROIDS]
            m_b = jnp.min(d2_b, axis=1, keepdims=True)
            hot_parts.append((d2_b <= m_b).astype(jnp.bfloat16))
        onehot = jnp.concatenate(hot_parts, axis=1)
        fused = jax.lax.dot_general(
            onehot, wda, (((1,), (0,)), ((), ())),
            precision=_DEFAULT, preferred_element_type=jnp.float32)
        khat_ref[rows, :] = fused[:, :HEAD_DIM]
        idx_ref[rows, :] = fused[:, HEAD_DIM:].astype(jnp.int32)


def kernel(k, E_blocks, centroids, decoded_centroids):
    batch_shape = k.shape[:-1]
    kf = k.reshape(-1, HEAD_DIM).astype(jnp.float32)
    n = kf.shape[0]

    eye = jnp.eye(N_BLOCKS, dtype=jnp.float32)
    we = (eye[:, None, :, None] * jnp.swapaxes(E_blocks, 1, 2)[:, :, None, :]
          ).reshape(HEAD_DIM, HEAD_DIM).astype(jnp.bfloat16)
    wcn = (eye[:, None, :, None]
           * (-2.0 * jnp.swapaxes(centroids, 1, 2))[:, :, None, :]
           ).reshape(HEAD_DIM, NC_ALL).astype(jnp.bfloat16)
    wd = (eye[:, None, :, None] * decoded_centroids[:, :, None, :]
          ).reshape(NC_ALL, HEAD_DIM)
    wi = (eye[:, None, :]
          * jnp.arange(N_CENTROIDS, dtype=jnp.float32)[None, :, None]
          ).reshape(NC_ALL, N_BLOCKS)
    wda = jnp.concatenate([wd, wi], axis=1).astype(jnp.bfloat16)
    c2 = jnp.sum(centroids * centroids, axis=-1).reshape(1, NC_ALL)

    tile = 2048
    grid = (n // tile,)
    idx, khat = pl.pallas_call(
        _vq_body,
        grid=grid,
        in_specs=[
            pl.BlockSpec((tile, HEAD_DIM), lambda i: (i, 0)),
            pl.BlockSpec((HEAD_DIM, HEAD_DIM), lambda i: (0, 0)),
            pl.BlockSpec((HEAD_DIM, NC_ALL), lambda i: (0, 0)),
            pl.BlockSpec((NC_ALL, HEAD_DIM + N_BLOCKS), lambda i: (0, 0)),
            pl.BlockSpec((1, NC_ALL), lambda i: (0, 0)),
        ],
        out_specs=[
            pl.BlockSpec((tile, N_BLOCKS), lambda i: (i, 0)),
            pl.BlockSpec((tile, HEAD_DIM), lambda i: (i, 0)),
        ],
        out_shape=[
            jax.ShapeDtypeStruct((n, N_BLOCKS), jnp.int32),
            jax.ShapeDtypeStruct((n, HEAD_DIM), jnp.float32),
        ],
    )(kf, we, wcn, wda, c2)

    return (idx.reshape(*batch_shape, N_BLOCKS),
            khat.reshape(*batch_shape, HEAD_DIM))
